# baseline (speedup 1.0000x reference)
import jax
import jax.numpy as jnp
from jax.experimental import pallas as pl
from jax.experimental.pallas import tpu as pltpu

B, N, M, D = 4, 4096, 4096, 3
TN = 512
ST = N // TN
G = 2
BP = B // G


def _chamfer_body(x_ref, y_ref, loss_ref, buf0, buf1):
    bufs = (buf0, buf1)
    acc = None
    for b in range(BP):
        yb = y_ref[b]
        col_min = None
        row_sum = None
        for i in range(ST):
            buf = bufs[i % 2]
            xi = x_ref[b, :, i * TN:(i + 1) * TN]
            buf[...] = jax.lax.dot_general(
                xi, yb, (((0,), (0,)), ((), ())),
                preferred_element_type=jnp.float32)
            dj = buf[...]
            rm = jnp.min(dj, axis=1)
            row_sum = rm if row_sum is None else row_sum + rm
            pm = jnp.min(dj, axis=0)
            col_min = pm if col_min is None else jnp.minimum(col_min, pm)
        s = jnp.sum(row_sum) + jnp.sum(col_min)
        acc = s if acc is None else acc + s

    loss_ref[0, 0, 0] = acc * (1.0 / (B * N))


def kernel(pred_pc, gt_pc):
    x2 = jnp.sum(pred_pc * pred_pc, axis=-1, keepdims=True)
    y2 = jnp.sum(gt_pc * gt_pc, axis=-1, keepdims=True)

    def split_hi_lo(v):
        hi = jax.lax.bitcast_convert_type(
            jax.lax.bitcast_convert_type(v, jnp.uint32) & jnp.uint32(0xFFFF0000),
            jnp.float32)
        return hi, v - hi

    x2h, x2l = split_hi_lo(x2)
    y2h, y2l = split_hi_lo(y2)
    ones_x = jnp.ones_like(x2)
    ones_y = jnp.ones_like(y2)
    zeros_x = jnp.zeros_like(x2)
    zeros_y = jnp.zeros_like(y2)
    xa = jnp.concatenate(
        [-2.0 * pred_pc, x2h, x2l, ones_x, ones_x, zeros_x],
        axis=-1)
    ya = jnp.concatenate(
        [gt_pc, ones_y, ones_y, y2h, y2l, zeros_y], axis=-1)
    xa_t = jnp.swapaxes(xa, 1, 2)
    ya_t = jnp.swapaxes(ya, 1, 2)

    partial = pl.pallas_call(
        _chamfer_body,
        grid=(G,),
        in_specs=[
            pl.BlockSpec((BP, 8, N), lambda g: (g, 0, 0),
                         memory_space=pltpu.VMEM),
            pl.BlockSpec((BP, 8, M), lambda g: (g, 0, 0),
                         memory_space=pltpu.VMEM),
        ],
        out_specs=pl.BlockSpec((1, 1, 1), lambda g: (g, 0, 0),
                               memory_space=pltpu.SMEM),
        out_shape=jax.ShapeDtypeStruct((G, 1, 1), jnp.float32),
        scratch_shapes=[
            pltpu.VMEM((TN, M), jnp.float32),
            pltpu.VMEM((TN, M), jnp.float32),
        ],
        compiler_params=pltpu.CompilerParams(
            dimension_semantics=("parallel",),
        ),
    )(xa_t, ya_t)
    return jnp.sum(partial)


# speedup vs baseline: 1.0159x; 1.0159x over previous
import jax
import jax.numpy as jnp
from jax.experimental import pallas as pl
from jax.experimental.pallas import tpu as pltpu

B, N, M, D = 4, 4096, 4096, 3
TN = 512
ST = N // TN


def _chamfer_body(x_ref, y_ref, loss_ref, buf0, buf1):
    bufs = (buf0, buf1)
    acc = None
    for b in range(B):
        yb = y_ref[b]
        col_min = None
        row_sum = None
        for i in range(ST):
            buf = bufs[i % 2]
            xi = x_ref[b, :, i * TN:(i + 1) * TN]
            buf[...] = jax.lax.dot_general(
                xi, yb, (((0,), (0,)), ((), ())),
                preferred_element_type=jnp.float32,
                ).astype(jnp.bfloat16)
            dj = buf[...]
            rm = jnp.min(dj, axis=1)
            rm32 = rm.astype(jnp.float32)
            row_sum = rm32 if row_sum is None else row_sum + rm32
            pm = jnp.min(dj, axis=0)
            col_min = pm if col_min is None else jnp.minimum(col_min, pm)
        s = jnp.sum(row_sum) + jnp.sum(col_min.astype(jnp.float32))
        acc = s if acc is None else acc + s

    loss_ref[0, 0] = acc * (1.0 / (B * N))


def kernel(pred_pc, gt_pc):
    x2 = jnp.sum(pred_pc * pred_pc, axis=-1, keepdims=True)
    y2 = jnp.sum(gt_pc * gt_pc, axis=-1, keepdims=True)

    def split_hi_lo(v):
        hi = jax.lax.bitcast_convert_type(
            jax.lax.bitcast_convert_type(v, jnp.uint32) & jnp.uint32(0xFFFF0000),
            jnp.float32)
        return hi, v - hi

    x2h, x2l = split_hi_lo(x2)
    y2h, y2l = split_hi_lo(y2)
    ones_x = jnp.ones_like(x2)
    ones_y = jnp.ones_like(y2)
    zeros_x = jnp.zeros_like(x2)
    zeros_y = jnp.zeros_like(y2)
    xa = jnp.concatenate(
        [-2.0 * pred_pc, x2h, x2l, ones_x, ones_x, zeros_x],
        axis=-1)
    ya = jnp.concatenate(
        [gt_pc, ones_y, ones_y, y2h, y2l, zeros_y], axis=-1)
    xa_t = jnp.swapaxes(xa, 1, 2)
    ya_t = jnp.swapaxes(ya, 1, 2)

    loss = pl.pallas_call(
        _chamfer_body,
        in_specs=[
            pl.BlockSpec(memory_space=pltpu.VMEM),
            pl.BlockSpec(memory_space=pltpu.VMEM),
        ],
        out_specs=pl.BlockSpec(memory_space=pltpu.SMEM),
        out_shape=jax.ShapeDtypeStruct((1, 1), jnp.float32),
        scratch_shapes=[
            pltpu.VMEM((TN, M), jnp.bfloat16),
            pltpu.VMEM((TN, M), jnp.bfloat16),
        ],
    )(xa_t, ya_t)
    return loss[0, 0]
